# Initial kernel scaffold; baseline (speedup 1.0000x reference)
import jax
import jax.numpy as jnp
from jax.experimental import pallas as pl


def kernel(logits, target):
    raise NotImplementedError("write your pallas kernel here")


# speedup vs baseline: 5.0595x; 5.0595x over previous
import jax
import jax.numpy as jnp
from jax import lax
from jax.experimental import pallas as pl
from jax.experimental.pallas import tpu as pltpu

_BINS = 30
_SCALE = 30 - 0.0001
_LANES = 128


def _bins(x, t):
    g = jnp.abs(jax.nn.sigmoid(x) - t)
    return jnp.floor(g * _SCALE).astype(jnp.int32)


def _hist_kernel(x_ref, t_ref, hist_ref):
    @pl.when(pl.program_id(0) == 0)
    def _init():
        hist_ref[...] = jnp.zeros_like(hist_ref)

    b = _bins(x_ref[...], t_ref[...])
    li = lax.broadcasted_iota(jnp.int32, (1, _LANES), 1)
    vec = jnp.zeros((1, _LANES), jnp.float32)
    for k in range(_BINS):
        c = jnp.sum((b == k).astype(jnp.float32))
        vec = vec + jnp.where(li == k, c, 0.0)
    hist_ref[...] += vec


def _loss_kernel(hist_ref, x_ref, t_ref, out_ref, *, tot):
    cnt = hist_ref[...]
    li = lax.broadcasted_iota(jnp.int32, (1, _LANES), 1)
    ne = jnp.sum(jnp.where((li < _BINS) & (cnt > 0), 1.0, 0.0))
    beta = tot / jnp.clip(cnt * ne, 0.0001, None)

    x = x_ref[...]
    t = t_ref[...]
    b = _bins(x, t)
    w = jnp.zeros_like(x)
    for k in range(_BINS):
        w = jnp.where(b == k, beta[0, k], w)
    loss = w * (jnp.maximum(x, 0.0) - x * t + jnp.log1p(jnp.exp(-jnp.abs(x))))
    out_ref[...] = jnp.mean(loss, axis=1)


def kernel(logits, target):
    rows, cols = logits.shape
    br = 512
    grid = (rows // br,)
    tot = float(logits.size)

    hist = pl.pallas_call(
        _hist_kernel,
        grid=grid,
        in_specs=[
            pl.BlockSpec((br, cols), lambda i: (i, 0)),
            pl.BlockSpec((br, cols), lambda i: (i, 0)),
        ],
        out_specs=pl.BlockSpec((1, _LANES), lambda i: (0, 0)),
        out_shape=jax.ShapeDtypeStruct((1, _LANES), jnp.float32),
        compiler_params=pltpu.CompilerParams(
            dimension_semantics=("arbitrary",),
        ),
    )(logits, target)

    import functools

    out = pl.pallas_call(
        functools.partial(_loss_kernel, tot=tot),
        grid=grid,
        in_specs=[
            pl.BlockSpec((1, _LANES), lambda i: (0, 0)),
            pl.BlockSpec((br, cols), lambda i: (i, 0)),
            pl.BlockSpec((br, cols), lambda i: (i, 0)),
        ],
        out_specs=pl.BlockSpec((br,), lambda i: (i,)),
        out_shape=jax.ShapeDtypeStruct((rows,), jnp.float32),
        compiler_params=pltpu.CompilerParams(
            dimension_semantics=("arbitrary",),
        ),
    )(hist, logits, target)
    return out


# speedup vs baseline: 19.0136x; 3.7580x over previous
import functools

import jax
import jax.numpy as jnp
from jax import lax
from jax.experimental import pallas as pl
from jax.experimental.pallas import tpu as pltpu

_BINS = 30
_SCALE = 30 - 0.0001
_LANES = 128
_BR = 512
_CH = 8
_LEVELS = 12


def _bins(x, t):
    g = jnp.abs(jax.nn.sigmoid(x) - t)
    return jnp.floor(g * _SCALE).astype(jnp.int32)


def _csa(a, b, c):
    u = a ^ b
    return u ^ c, (a & b) | (u & c)


def _hist_kernel(x_ref, t_ref, hist_ref, planes_ref, *, nblocks):
    i = pl.program_id(0)

    @pl.when(i == 0)
    def _init():
        planes_ref[...] = jnp.zeros_like(planes_ref)

    b = _bins(x_ref[...], t_ref[...])
    v = jnp.left_shift(jnp.int32(1), b)

    vals = {0: [v[k * _CH:(k + 1) * _CH, :] for k in range(_BR // _CH)]}
    j = 0
    while j in vals:
        lv = vals[j]
        carries = []
        while len(lv) >= 3:
            s, co = _csa(lv.pop(), lv.pop(), lv.pop())
            lv.append(s)
            carries.append(co)
        if len(lv) == 2:
            a0, a1 = lv
            lv = [a0 ^ a1]
            carries.append(a0 & a1)
        if carries:
            vals[j + 1] = carries
        if lv:
            carry = lv[0]
            for lvl in range(j, _LEVELS):
                old = planes_ref[lvl]
                planes_ref[lvl] = old ^ carry
                carry = old & carry
        j += 1

    @pl.when(i == nblocks - 1)
    def _extract():
        li = lax.broadcasted_iota(jnp.int32, (1, _LANES), 1)
        vec = jnp.zeros((1, _LANES), jnp.float32)
        for k in range(_BINS):
            c = jnp.float32(0.0)
            for lvl in range(_LEVELS):
                bits = (planes_ref[lvl] >> k) & 1
                c = c + jnp.float32(1 << lvl) * jnp.sum(bits).astype(jnp.float32)
            vec = vec + jnp.where(li == k, c, 0.0)
        hist_ref[...] = vec


def _loss_kernel(hist_ref, x_ref, t_ref, out_ref, *, tot):
    cnt = hist_ref[...]
    li = lax.broadcasted_iota(jnp.int32, (1, _LANES), 1)
    ne = jnp.sum(jnp.where((li < _BINS) & (cnt > 0), 1.0, 0.0))
    beta = tot / jnp.clip(cnt * ne, 0.0001, None)

    x = x_ref[...]
    t = t_ref[...]
    b = _bins(x, t)
    tab = jnp.broadcast_to(beta[:, :32], (x.shape[0], 32))
    w = jnp.take_along_axis(tab, b, axis=1)
    loss = w * (jnp.maximum(x, 0.0) - x * t + jnp.log1p(jnp.exp(-jnp.abs(x))))
    out_ref[...] = jnp.mean(loss, axis=1)


def kernel(logits, target):
    rows, cols = logits.shape
    nblocks = rows // _BR
    tot = float(logits.size)

    hist = pl.pallas_call(
        functools.partial(_hist_kernel, nblocks=nblocks),
        grid=(nblocks,),
        in_specs=[
            pl.BlockSpec((_BR, cols), lambda i: (i, 0)),
            pl.BlockSpec((_BR, cols), lambda i: (i, 0)),
        ],
        out_specs=pl.BlockSpec((1, _LANES), lambda i: (0, 0)),
        out_shape=jax.ShapeDtypeStruct((1, _LANES), jnp.float32),
        scratch_shapes=[pltpu.VMEM((_LEVELS, _CH, cols), jnp.int32)],
        compiler_params=pltpu.CompilerParams(
            dimension_semantics=("arbitrary",),
        ),
    )(logits, target)

    out = pl.pallas_call(
        functools.partial(_loss_kernel, tot=tot),
        grid=(nblocks,),
        in_specs=[
            pl.BlockSpec((1, _LANES), lambda i: (0, 0)),
            pl.BlockSpec((_BR, cols), lambda i: (i, 0)),
            pl.BlockSpec((_BR, cols), lambda i: (i, 0)),
        ],
        out_specs=pl.BlockSpec((_BR,), lambda i: (i,)),
        out_shape=jax.ShapeDtypeStruct((rows,), jnp.float32),
        compiler_params=pltpu.CompilerParams(
            dimension_semantics=("arbitrary",),
        ),
    )(hist, logits, target)
    return out


# speedup vs baseline: 19.5509x; 1.0283x over previous
import functools

import jax
import jax.numpy as jnp
from jax import lax
from jax.experimental import pallas as pl
from jax.experimental.pallas import tpu as pltpu

_BINS = 30
_SCALE = 30 - 0.0001
_LANES = 128
_BR = 1024
_CH = 8
_LEVELS = 12


def _bins(x, t):
    g = jnp.abs(jax.nn.sigmoid(x) - t)
    return jnp.floor(g * _SCALE).astype(jnp.int32)


def _csa(a, b, c):
    u = a ^ b
    return u ^ c, (a & b) | (u & c)


def _hist_kernel(x_ref, t_ref, hist_ref, planes_ref, *, nblocks):
    i = pl.program_id(0)

    @pl.when(i == 0)
    def _init():
        planes_ref[...] = jnp.zeros_like(planes_ref)

    b = _bins(x_ref[...], t_ref[...])
    v = jnp.left_shift(jnp.int32(1), b)

    vals = {0: [v[k * _CH:(k + 1) * _CH, :] for k in range(_BR // _CH)]}
    j = 0
    while j in vals:
        lv = vals[j]
        carries = []
        while len(lv) >= 3:
            s, co = _csa(lv.pop(), lv.pop(), lv.pop())
            lv.append(s)
            carries.append(co)
        if len(lv) == 2:
            a0, a1 = lv
            lv = [a0 ^ a1]
            carries.append(a0 & a1)
        if carries:
            vals[j + 1] = carries
        if lv:
            carry = lv[0]
            for lvl in range(j, _LEVELS):
                old = planes_ref[lvl]
                planes_ref[lvl] = old ^ carry
                carry = old & carry
        j += 1

    @pl.when(i == nblocks - 1)
    def _extract():
        li = lax.broadcasted_iota(jnp.int32, (1, _LANES), 1)
        vec = jnp.zeros((1, _LANES), jnp.float32)
        for k in range(_BINS):
            c = jnp.float32(0.0)
            for lvl in range(_LEVELS):
                bits = (planes_ref[lvl] >> k) & 1
                c = c + jnp.float32(1 << lvl) * jnp.sum(bits).astype(jnp.float32)
            vec = vec + jnp.where(li == k, c, 0.0)
        hist_ref[...] = vec


def _loss_kernel(hist_ref, x_ref, t_ref, out_ref, *, tot):
    cnt = hist_ref[...]
    li = lax.broadcasted_iota(jnp.int32, (1, _LANES), 1)
    ne = jnp.sum(jnp.where((li < _BINS) & (cnt > 0), 1.0, 0.0))
    beta = tot / jnp.clip(cnt * ne, 0.0001, None)

    x = x_ref[...]
    t = t_ref[...]
    b = _bins(x, t)
    tab = jnp.broadcast_to(beta[:, :32], (x.shape[0], 32))
    w = jnp.take_along_axis(tab, b, axis=1)
    loss = w * (jnp.maximum(x, 0.0) - x * t + jnp.log1p(jnp.exp(-jnp.abs(x))))
    out_ref[...] = jnp.mean(loss, axis=1)


def kernel(logits, target):
    rows, cols = logits.shape
    nblocks = rows // _BR
    tot = float(logits.size)

    hist = pl.pallas_call(
        functools.partial(_hist_kernel, nblocks=nblocks),
        grid=(nblocks,),
        in_specs=[
            pl.BlockSpec((_BR, cols), lambda i: (i, 0)),
            pl.BlockSpec((_BR, cols), lambda i: (i, 0)),
        ],
        out_specs=pl.BlockSpec((1, _LANES), lambda i: (0, 0)),
        out_shape=jax.ShapeDtypeStruct((1, _LANES), jnp.float32),
        scratch_shapes=[pltpu.VMEM((_LEVELS, _CH, cols), jnp.int32)],
        compiler_params=pltpu.CompilerParams(
            dimension_semantics=("arbitrary",),
        ),
    )(logits, target)

    out = pl.pallas_call(
        functools.partial(_loss_kernel, tot=tot),
        grid=(nblocks,),
        in_specs=[
            pl.BlockSpec((1, _LANES), lambda i: (0, 0)),
            pl.BlockSpec((_BR, cols), lambda i: (i, 0)),
            pl.BlockSpec((_BR, cols), lambda i: (i, 0)),
        ],
        out_specs=pl.BlockSpec((_BR,), lambda i: (i,)),
        out_shape=jax.ShapeDtypeStruct((rows,), jnp.float32),
        compiler_params=pltpu.CompilerParams(
            dimension_semantics=("arbitrary",),
        ),
    )(hist, logits, target)
    return out
